# baseline (speedup 1.0000x reference)
import jax
import jax.numpy as jnp
from jax import lax
from jax.experimental import pallas as pl
from jax.experimental.pallas import tpu as pltpu
from jax.experimental.pallas import tpu_sc as plsc

BASE_NUM = 64
VNUM = 50000
DIM = 9
K = 8
B = 16
ROW = B * DIM
RSTRIDE = ROW + 1
VB = 400
NBLK = VNUM // VB
NWORKERS = 32
OBN = VB * DIM
CCHUNK = 12


def _sc_body(t_hbm, ws_hbm, vb_hbm, out_hbm, tbl, wv, ib, ob, sem):
    c = lax.axis_index("c")
    s = lax.axis_index("s")
    wid = s * 2 + c

    pltpu.sync_copy(t_hbm, tbl)

    lane = lax.broadcasted_iota(jnp.int32, (16,), 0)
    lane8 = lane * K
    lane9 = lane * DIM

    def blk_body(i, _):
        blk = wid + i * NWORKERS
        v0 = blk * VB
        pltpu.sync_copy(ws_hbm.at[pl.ds(v0 * K, VB * K)], wv)
        pltpu.sync_copy(vb_hbm.at[pl.ds(v0 * K, VB * K)], ib)

        def g_body(g, _):
            gidx = lane8 + g * (16 * K)
            w = [plsc.load_gather(wv, [gidx + k]) for k in range(K)]
            ov = [
                plsc.load_gather(ib, [gidx + k]) * RSTRIDE for k in range(K)
            ]
            m = w[0]
            for k in range(1, K):
                m = jnp.maximum(m, w[k])
            e = [jnp.exp(wk - m) for wk in w]
            ssum = e[0]
            for k in range(1, K):
                ssum = ssum + e[k]
            inv = 1.0 / (ssum + 1e-16)
            p = [ek * inv for ek in e]

            sbase = lane9 + g * (16 * DIM)
            for c0 in range(0, ROW, CCHUNK):
                accs = [
                    plsc.load_gather(tbl, [ov[0] + (c0 + ci)]) * p[0]
                    for ci in range(CCHUNK)
                ]
                for k in range(1, K):
                    for ci in range(CCHUNK):
                        accs[ci] = accs[ci] + (
                            plsc.load_gather(tbl, [ov[k] + (c0 + ci)]) * p[k]
                        )
                for ci in range(CCHUNK):
                    col = c0 + ci
                    scol = (col % 16) * OBN + col // 16
                    plsc.store_scatter(ob, [sbase + scol], accs[ci])
            return 0

        lax.fori_loop(0, VB // 16, g_body, 0)

        cps = [
            pltpu.async_copy(
                ob.at[pl.ds(b * OBN, OBN)],
                out_hbm.at[pl.ds(b * (VNUM * DIM) + v0 * DIM, OBN)],
                sem,
            )
            for b in range(B)
        ]
        for cp in cps:
            cp.wait()
        return 0

    nblk = (NBLK - wid + NWORKERS - 1) // NWORKERS
    lax.fori_loop(0, nblk, blk_body, 0)


@jax.jit
def _skin(t2, ws, vb):
    run = pl.kernel(
        _sc_body,
        out_type=jax.ShapeDtypeStruct((B * VNUM * DIM,), jnp.float32),
        mesh=plsc.VectorSubcoreMesh(core_axis_name="c", subcore_axis_name="s"),
        scratch_types=[
            pltpu.VMEM((BASE_NUM * RSTRIDE,), jnp.float32),
            pltpu.VMEM((VB * K,), jnp.float32),
            pltpu.VMEM((VB * K,), jnp.int32),
            pltpu.VMEM((B * OBN,), jnp.float32),
            pltpu.SemaphoreType.DMA,
        ],
        compiler_params=pltpu.CompilerParams(needs_layout_passes=False),
    )
    return run(t2, ws, vb)


def kernel(base_fs, ws, vb_index, vv_index):
    t2 = base_fs.reshape(B, BASE_NUM, DIM).transpose(1, 2, 0).reshape(BASE_NUM, ROW)
    t2p = jnp.pad(t2, ((0, 0), (0, RSTRIDE - ROW))).reshape(-1)
    out = _skin(t2p, ws, vb_index)
    return out.reshape(B, VNUM, DIM)


# speedup vs baseline: 4.5328x; 4.5328x over previous
import jax
import jax.numpy as jnp
from jax import lax
from jax.experimental import pallas as pl
from jax.experimental.pallas import tpu as pltpu
from jax.experimental.pallas import tpu_sc as plsc

BASE_NUM = 64
VNUM = 50000
VNUMP = 50048
DIM = 9
K = 8
B = 16
ROW = B * DIM
RSTRIDE = ROW + 1
VB = 128
NBLK = VNUMP // VB
NWORKERS = 32
CCHUNK = 12


def _sc_body(t_hbm, ws_hbm, vb_hbm, out_hbm, tbl, wv, ib, ob, sem):
    c = lax.axis_index("c")
    s = lax.axis_index("s")
    wid = s * 2 + c

    pltpu.sync_copy(t_hbm, tbl)

    lane = lax.broadcasted_iota(jnp.int32, (16,), 0)
    lane8 = lane * K

    def blk_body(i, _):
        blk = wid + i * NWORKERS
        v0 = blk * VB
        pltpu.sync_copy(ws_hbm.at[pl.ds(v0 * K, VB * K)], wv)
        pltpu.sync_copy(vb_hbm.at[pl.ds(v0 * K, VB * K)], ib)

        def g_body(g, _):
            gidx = lane8 + g * (16 * K)
            w = [plsc.load_gather(wv, [gidx + k]) for k in range(K)]
            ov = [
                plsc.load_gather(ib, [gidx + k]) * RSTRIDE for k in range(K)
            ]
            m = w[0]
            for k in range(1, K):
                m = jnp.maximum(m, w[k])
            e = [jnp.exp(wk - m) for wk in w]
            ssum = e[0]
            for k in range(1, K):
                ssum = ssum + e[k]
            inv = 1.0 / (ssum + 1e-16)
            p = [ek * inv for ek in e]

            vl = g * 16
            for c0 in range(0, ROW, CCHUNK):
                accs = [
                    plsc.load_gather(tbl, [ov[0] + (c0 + ci)]) * p[0]
                    for ci in range(CCHUNK)
                ]
                for k in range(1, K):
                    for ci in range(CCHUNK):
                        accs[ci] = accs[ci] + (
                            plsc.load_gather(tbl, [ov[k] + (c0 + ci)]) * p[k]
                        )
                for ci in range(CCHUNK):
                    col = c0 + ci
                    ob[col // 16, col % 16, pl.ds(vl, 16)] = accs[ci]
            return 0

        lax.fori_loop(0, VB // 16, g_body, 0)

        cps = [
            pltpu.async_copy(
                ob.at[d],
                out_hbm.at[d, :, pl.ds(v0, VB)],
                sem,
            )
            for d in range(DIM)
        ]
        for cp in cps:
            cp.wait()
        return 0

    nblk = (NBLK - wid + NWORKERS - 1) // NWORKERS
    lax.fori_loop(0, nblk, blk_body, 0)


@jax.jit
def _skin(t2, ws, vb):
    run = pl.kernel(
        _sc_body,
        out_type=jax.ShapeDtypeStruct((DIM, B, VNUMP), jnp.float32),
        mesh=plsc.VectorSubcoreMesh(core_axis_name="c", subcore_axis_name="s"),
        scratch_types=[
            pltpu.VMEM((BASE_NUM * RSTRIDE,), jnp.float32),
            pltpu.VMEM((VB * K,), jnp.float32),
            pltpu.VMEM((VB * K,), jnp.int32),
            pltpu.VMEM((DIM, B, VB), jnp.float32),
            pltpu.SemaphoreType.DMA,
        ],
        compiler_params=pltpu.CompilerParams(needs_layout_passes=False),
    )
    return run(t2, ws, vb)


def kernel(base_fs, ws, vb_index, vv_index):
    t2 = base_fs.reshape(B, BASE_NUM, DIM).transpose(1, 2, 0).reshape(BASE_NUM, ROW)
    t2p = jnp.pad(t2, ((0, 0), (0, RSTRIDE - ROW))).reshape(-1)
    pad_e = (VNUMP - VNUM) * K
    wsp = jnp.pad(ws, (0, pad_e))
    vbp = jnp.pad(vb_index, (0, pad_e))
    out = _skin(t2p, wsp, vbp)
    return out[:, :, :VNUM].transpose(1, 2, 0)
